# baseline (speedup 1.0000x reference)
import math

import jax
import jax.numpy as jnp
from jax import lax
from jax.experimental import pallas as pl
from jax.experimental.pallas import tpu as pltpu
from jax.experimental.pallas import tpu_sc as plsc

B, S, H, V, E, R = 1, 2048, 1024, 32000, 8, 4
SCALING = 1.0 / R
RSQRT_H = 1.0 / math.sqrt(H)

S_TILE = 256
N_S = S // S_TILE
V_TILE = 1280
N_V = V // V_TILE

BF16 = jnp.bfloat16
F32 = jnp.float32

_NC, _NS = 2, 16
_NW = _NC * _NS
_BPW = S // _NW


def _gather_body(table_hbm, idx_hbm, out_hbm, idx_v, rows_v, sem):
    wid = lax.axis_index("s") * _NC + lax.axis_index("c")
    base = wid * _BPW
    pltpu.sync_copy(idx_hbm.at[pl.ds(base, _BPW)], idx_v)
    pltpu.async_copy(table_hbm.at[idx_v], rows_v, sem).wait()
    pltpu.sync_copy(rows_v, out_hbm.at[pl.ds(base, _BPW)])


def _sc_gather(emb, idx):
    mesh = plsc.VectorSubcoreMesh(core_axis_name="c", subcore_axis_name="s")
    return pl.kernel(
        _gather_body,
        mesh=mesh,
        out_type=jax.ShapeDtypeStruct((S, H), F32),
        scratch_types=[
            pltpu.VMEM((_BPW,), jnp.int32),
            pltpu.VMEM((_BPW, H), F32),
            pltpu.SemaphoreType.DMA,
        ],
    )(emb, idx)


def _qk_body(h_ref, Wq_ref, bq_ref, Aq_ref, Bq_ref, Wk_ref, bk_ref, Ak_ref,
             Bk_ref, Wr_ref, q_ref, k_ref, w_ref):
    h = h_ref[...]
    hb = h.astype(BF16)
    q = jnp.dot(hb, Wq_ref[...], preferred_element_type=F32)
    qa = jnp.dot(hb, Aq_ref[...], preferred_element_type=F32).astype(BF16)
    q += jnp.dot(qa, Bq_ref[...], preferred_element_type=F32) * SCALING
    q_ref[...] = (q + bq_ref[...]).astype(BF16)
    k = jnp.dot(hb, Wk_ref[...], preferred_element_type=F32)
    ka = jnp.dot(hb, Ak_ref[...], preferred_element_type=F32).astype(BF16)
    k += jnp.dot(ka, Bk_ref[...], preferred_element_type=F32) * SCALING
    k_ref[...] = (k + bk_ref[...]).astype(BF16)
    rl = jnp.dot(h, Wr_ref[...], preferred_element_type=F32)
    eidx = lax.broadcasted_iota(jnp.int32, (S_TILE, E), 1)
    m1 = jnp.max(rl, axis=-1, keepdims=True)
    i1 = jnp.min(jnp.where(rl == m1, eidx, E), axis=-1, keepdims=True)
    rl2 = jnp.where(eidx == i1, -jnp.inf, rl)
    m2 = jnp.max(rl2, axis=-1, keepdims=True)
    i2 = jnp.min(jnp.where(rl2 == m2, eidx, E), axis=-1, keepdims=True)
    t = jnp.exp(m2 - m1)
    w1 = 1.0 / (1.0 + t)
    w2 = w1 * t
    w_ref[...] = jnp.where(eidx == i1, w1, 0.0) + jnp.where(eidx == i2, w2, 0.0)


def _qk_router(h, Wq, bq, Aq, Bq, Wk, bk, Ak, Bk, Wr):
    return pl.pallas_call(
        _qk_body,
        grid=(N_S,),
        in_specs=[
            pl.BlockSpec((S_TILE, H), lambda i: (i, 0)),
            pl.BlockSpec((H, H), lambda i: (0, 0)),
            pl.BlockSpec((H,), lambda i: (0,)),
            pl.BlockSpec((H, R), lambda i: (0, 0)),
            pl.BlockSpec((R, H), lambda i: (0, 0)),
            pl.BlockSpec((H, H), lambda i: (0, 0)),
            pl.BlockSpec((H,), lambda i: (0,)),
            pl.BlockSpec((H, R), lambda i: (0, 0)),
            pl.BlockSpec((R, H), lambda i: (0, 0)),
            pl.BlockSpec((H, E), lambda i: (0, 0)),
        ],
        out_specs=[
            pl.BlockSpec((S_TILE, H), lambda i: (i, 0)),
            pl.BlockSpec((S_TILE, H), lambda i: (i, 0)),
            pl.BlockSpec((S_TILE, E), lambda i: (i, 0)),
        ],
        out_shape=[
            jax.ShapeDtypeStruct((S, H), BF16),
            jax.ShapeDtypeStruct((S, H), BF16),
            jax.ShapeDtypeStruct((S, E), F32),
        ],
    )(h, Wq.astype(BF16), bq, Aq.astype(BF16), Bq.astype(BF16),
      Wk.astype(BF16), bk, Ak.astype(BF16), Bk.astype(BF16), Wr)


def _att_body(q_ref, k_ref, imp_ref):
    att = lax.dot_general(q_ref[...], k_ref[...], (((1,), (1,)), ((), ())),
                          preferred_element_type=F32) * RSQRT_H
    m = jnp.max(att, axis=-1, keepdims=True)
    p = jnp.exp(att - m)
    probs = p / jnp.sum(p, axis=-1, keepdims=True)
    colsum = jnp.sum(probs, axis=0, keepdims=True)

    @pl.when(pl.program_id(0) == 0)
    def _():
        imp_ref[...] = colsum

    @pl.when(pl.program_id(0) != 0)
    def _():
        imp_ref[...] += colsum


def _attention(q, k):
    return pl.pallas_call(
        _att_body,
        grid=(N_S,),
        in_specs=[
            pl.BlockSpec((S_TILE, H), lambda i: (i, 0)),
            pl.BlockSpec((S, H), lambda i: (0, 0)),
        ],
        out_specs=pl.BlockSpec((1, S), lambda i: (0, 0)),
        out_shape=jax.ShapeDtypeStruct((1, S), F32),
    )(q, k)


def _moe_body(h_ref, We_ref, be_ref, Ae_ref, Be_ref, w_ref, out_ref, acc_ref):
    e = pl.program_id(1)
    hb = h_ref[...].astype(BF16)
    mm = jnp.dot(hb, We_ref[0], preferred_element_type=F32)
    la = jnp.dot(hb, Ae_ref[0], preferred_element_type=F32).astype(BF16)
    mm += jnp.dot(la, Be_ref[0], preferred_element_type=F32) * SCALING
    mm += be_ref[0]
    sel = (lax.broadcasted_iota(jnp.int32, (S_TILE, E), 1) == e)
    wcol = jnp.sum(w_ref[...] * sel.astype(F32), axis=1, keepdims=True)
    contrib = mm * wcol

    @pl.when(e == 0)
    def _():
        acc_ref[...] = contrib

    @pl.when(e != 0)
    def _():
        acc_ref[...] += contrib

    @pl.when(e == E - 1)
    def _():
        out_ref[...] = acc_ref[...].astype(BF16)


def _moe(h, We, be, Ae, Be, w):
    return pl.pallas_call(
        _moe_body,
        grid=(N_S, E),
        in_specs=[
            pl.BlockSpec((S_TILE, H), lambda s, e: (s, 0)),
            pl.BlockSpec((1, H, H), lambda s, e: (e, 0, 0)),
            pl.BlockSpec((1, 1, H), lambda s, e: (e, 0, 0)),
            pl.BlockSpec((1, H, R), lambda s, e: (e, 0, 0)),
            pl.BlockSpec((1, R, H), lambda s, e: (e, 0, 0)),
            pl.BlockSpec((S_TILE, E), lambda s, e: (s, 0)),
        ],
        out_specs=pl.BlockSpec((S_TILE, H), lambda s, e: (s, 0)),
        out_shape=jax.ShapeDtypeStruct((S, H), BF16),
        scratch_shapes=[pltpu.VMEM((S_TILE, H), F32)],
    )(h, We.astype(BF16), be.reshape(E, 1, H), Ae.astype(BF16),
      Be.astype(BF16), w)


def _vocab_body(moe_ref, Wv_ref, bv_ref, out_ref):
    out_ref[...] = (jnp.dot(moe_ref[...], Wv_ref[...],
                            preferred_element_type=F32) + bv_ref[...])


def _vocab(moe, Wv, bv):
    return pl.pallas_call(
        _vocab_body,
        grid=(N_V,),
        in_specs=[
            pl.BlockSpec((S, H), lambda j: (0, 0)),
            pl.BlockSpec((H, V_TILE), lambda j: (0, j)),
            pl.BlockSpec((1, V_TILE), lambda j: (0, j)),
        ],
        out_specs=pl.BlockSpec((S, V_TILE), lambda j: (0, j)),
        out_shape=jax.ShapeDtypeStruct((S, V), F32),
    )(moe, Wv.astype(BF16), bv.reshape(1, V))


def kernel(x, emb, Wq, bq, Aq, Bq, Wk, bk, Ak, Bk, Wr, We, be, Ae, Be, Wv, bv):
    idx = x.reshape(S).astype(jnp.int32)
    h = _sc_gather(emb, idx)
    q, k, w = _qk_router(h, Wq, bq, Aq, Bq, Wk, bk, Ak, Bk, Wr)
    imp = _attention(q, k)
    moe_out = _moe(h, We, be, Ae, Be, w)
    logits = _vocab(moe_out, Wv, bv)
    return (logits.reshape(B, S, V), imp)


# speedup vs baseline: 1.3215x; 1.3215x over previous
import math

import jax
import jax.numpy as jnp
from jax import lax
from jax.experimental import pallas as pl
from jax.experimental.pallas import tpu as pltpu
from jax.experimental.pallas import tpu_sc as plsc

B, S, H, V, E, R = 1, 2048, 1024, 32000, 8, 4
SCALING = 1.0 / R
RSQRT_H = 1.0 / math.sqrt(H)

S_TILE = 256
N_S = S // S_TILE
V_TILE = 1280
N_V = V // V_TILE

F32 = jnp.float32

_NC, _NS = 2, 16
_NW = _NC * _NS
_BPW = S // _NW


def _gather_body(table_hbm, idx_hbm, out_hbm, idx_v, rows_v, sem):
    wid = lax.axis_index("s") * _NC + lax.axis_index("c")
    base = wid * _BPW
    pltpu.sync_copy(idx_hbm.at[pl.ds(base, _BPW)], idx_v)
    pltpu.async_copy(table_hbm.at[idx_v], rows_v, sem).wait()
    pltpu.sync_copy(rows_v, out_hbm.at[pl.ds(base, _BPW)])


def _sc_gather(emb, idx):
    mesh = plsc.VectorSubcoreMesh(core_axis_name="c", subcore_axis_name="s")
    return pl.kernel(
        _gather_body,
        mesh=mesh,
        out_type=jax.ShapeDtypeStruct((S, H), F32),
        scratch_types=[
            pltpu.VMEM((_BPW,), jnp.int32),
            pltpu.VMEM((_BPW, H), F32),
            pltpu.SemaphoreType.DMA,
        ],
    )(emb, idx)


def _qkatt_body(h_ref, Wq_ref, bq_ref, Aq_ref, Bq_ref, Wk_ref, bk_ref,
                Ak_ref, Bk_ref, Wr_ref, w_ref, imp_ref, k_scr, att_scr):
    i = pl.program_id(0)

    @pl.when(i < N_S)
    def _():
        h = h_ref[...]
        k = jnp.dot(h, Wk_ref[...], preferred_element_type=F32)
        k += jnp.dot(jnp.dot(h, Ak_ref[...], preferred_element_type=F32),
                     Bk_ref[...], preferred_element_type=F32) * SCALING
        k_scr[pl.ds(i * S_TILE, S_TILE), :] = k + bk_ref[...]
        rl = jnp.dot(h, Wr_ref[...], preferred_element_type=F32)
        eidx = lax.broadcasted_iota(jnp.int32, (S_TILE, E), 1)
        m1 = jnp.max(rl, axis=-1, keepdims=True)
        i1 = jnp.min(jnp.where(rl == m1, eidx, E), axis=-1, keepdims=True)
        rl2 = jnp.where(eidx == i1, -jnp.inf, rl)
        m2 = jnp.max(rl2, axis=-1, keepdims=True)
        i2 = jnp.min(jnp.where(rl2 == m2, eidx, E), axis=-1, keepdims=True)
        t = jnp.exp(m2 - m1)
        w1 = 1.0 / (1.0 + t)
        w2 = w1 * t
        w_ref[...] = (jnp.where(eidx == i1, w1, 0.0) +
                      jnp.where(eidx == i2, w2, 0.0))

    @pl.when(jnp.logical_and(i >= N_S, i < 2 * N_S))
    def _():
        h = h_ref[...]
        q = jnp.dot(h, Wq_ref[...], preferred_element_type=F32)
        q += jnp.dot(jnp.dot(h, Aq_ref[...], preferred_element_type=F32),
                     Bq_ref[...], preferred_element_type=F32) * SCALING
        q += bq_ref[...]
        att = lax.dot_general(q, k_scr[...], (((1,), (1,)), ((), ())),
                              preferred_element_type=F32) * RSQRT_H
        att_scr[(i - N_S) % 2] = att

    @pl.when(i > N_S)
    def _():
        att = att_scr[(i - N_S - 1) % 2]
        m = jnp.max(att, axis=-1, keepdims=True)
        p = jnp.exp(att - m)
        r = 1.0 / jnp.sum(p, axis=-1, keepdims=True)
        colsum = jnp.sum(p * r, axis=0, keepdims=True)

        @pl.when(i == N_S + 1)
        def _():
            imp_ref[...] = colsum

        @pl.when(i > N_S + 1)
        def _():
            imp_ref[...] += colsum


def _qkatt(h, Wq, bq, Aq, Bq, Wk, bk, Ak, Bk, Wr):
    return pl.pallas_call(
        _qkatt_body,
        grid=(2 * N_S + 1,),
        in_specs=[
            pl.BlockSpec((S_TILE, H),
                         lambda i: (jnp.minimum(i, 2 * N_S - 1) % N_S, 0)),
            pl.BlockSpec((H, H), lambda i: (0, 0)),
            pl.BlockSpec((H,), lambda i: (0,)),
            pl.BlockSpec((H, R), lambda i: (0, 0)),
            pl.BlockSpec((R, H), lambda i: (0, 0)),
            pl.BlockSpec((H, H), lambda i: (0, 0)),
            pl.BlockSpec((H,), lambda i: (0,)),
            pl.BlockSpec((H, R), lambda i: (0, 0)),
            pl.BlockSpec((R, H), lambda i: (0, 0)),
            pl.BlockSpec((H, E), lambda i: (0, 0)),
        ],
        out_specs=[
            pl.BlockSpec((S_TILE, E), lambda i: (jnp.minimum(i, N_S - 1), 0)),
            pl.BlockSpec((1, S), lambda i: (0, 0)),
        ],
        out_shape=[
            jax.ShapeDtypeStruct((S, E), F32),
            jax.ShapeDtypeStruct((1, S), F32),
        ],
        scratch_shapes=[
            pltpu.VMEM((S, H), F32),
            pltpu.VMEM((2, S_TILE, S), F32),
        ],
    )(h, Wq, bq, Aq, Bq, Wk, bk, Ak, Bk, Wr)


def _moe_body(h_ref, We_ref, be_ref, Ae_ref, Be_ref, w_ref, out_ref):
    e = pl.program_id(0)
    h = h_ref[...]
    mm = jnp.dot(h, We_ref[0], preferred_element_type=F32)
    mm += jnp.dot(jnp.dot(h, Ae_ref[0], preferred_element_type=F32),
                  Be_ref[0], preferred_element_type=F32) * SCALING
    mm += be_ref[0]
    sel = (lax.broadcasted_iota(jnp.int32, (S, E), 1) == e)
    wcol = jnp.sum(w_ref[...] * sel.astype(F32), axis=1, keepdims=True)
    contrib = mm * wcol

    @pl.when(e == 0)
    def _():
        out_ref[...] = contrib

    @pl.when(e != 0)
    def _():
        out_ref[...] += contrib


def _moe(h, We, be, Ae, Be, w):
    return pl.pallas_call(
        _moe_body,
        grid=(E,),
        in_specs=[
            pl.BlockSpec((S, H), lambda e: (0, 0)),
            pl.BlockSpec((1, H, H), lambda e: (e, 0, 0)),
            pl.BlockSpec((1, 1, H), lambda e: (e, 0, 0)),
            pl.BlockSpec((1, H, R), lambda e: (e, 0, 0)),
            pl.BlockSpec((1, R, H), lambda e: (e, 0, 0)),
            pl.BlockSpec((S, E), lambda e: (0, 0)),
        ],
        out_specs=pl.BlockSpec((S, H), lambda e: (0, 0)),
        out_shape=jax.ShapeDtypeStruct((S, H), F32),
    )(h, We, be.reshape(E, 1, H), Ae, Be, w)


def _vocab_body(moe_ref, Wv_ref, bv_ref, out_ref):
    out_ref[...] = (jnp.dot(moe_ref[...], Wv_ref[...],
                            preferred_element_type=F32) + bv_ref[...])


def _vocab(moe, Wv, bv):
    return pl.pallas_call(
        _vocab_body,
        grid=(N_V,),
        in_specs=[
            pl.BlockSpec((S, H), lambda j: (0, 0)),
            pl.BlockSpec((H, V_TILE), lambda j: (0, j)),
            pl.BlockSpec((1, V_TILE), lambda j: (0, j)),
        ],
        out_specs=pl.BlockSpec((S, V_TILE), lambda j: (0, j)),
        out_shape=jax.ShapeDtypeStruct((S, V), F32),
    )(moe, Wv, bv.reshape(1, V))


def kernel(x, emb, Wq, bq, Aq, Bq, Wk, bk, Ak, Bk, Wr, We, be, Ae, Be, Wv, bv):
    idx = x.reshape(S).astype(jnp.int32)
    h = _sc_gather(emb, idx)
    w, imp = _qkatt(h, Wq, bq, Aq, Bq, Wk, bk, Ak, Bk, Wr)
    moe_out = _moe(h, We, be, Ae, Be, w)
    logits = _vocab(moe_out, Wv, bv)
    return (logits.reshape(B, S, V), imp)
